# baseline (speedup 1.0000x reference)
import functools

import jax
import jax.numpy as jnp
from jax.experimental import pallas as pl
from jax.experimental.pallas import tpu as pltpu


def _onehot_argmax_kernel(inv_tau_ref, a_ref, g_ref, o_ref, *, k):
    x = (a_ref[...] + g_ref[...]) * inv_tau_ref[0]
    m = jnp.max(x, axis=1, keepdims=True)
    ids = jax.lax.broadcasted_iota(jnp.int32, x.shape, 1)
    idx = jnp.min(jnp.where(x >= m, ids, jnp.int32(k)), axis=1, keepdims=True)
    o_ref[...] = (ids == idx).astype(o_ref.dtype)


def kernel(alpha, gumbel, tau):
    m, k = alpha.shape
    r = 8
    inv_tau = (jnp.float32(1.0) / jnp.asarray(tau, jnp.float32)).reshape(1)
    grid = (m // r,)
    out = pl.pallas_call(
        functools.partial(_onehot_argmax_kernel, k=k),
        grid=grid,
        in_specs=[
            pl.BlockSpec(memory_space=pltpu.SMEM),
            pl.BlockSpec((r, k), lambda i: (i, 0)),
            pl.BlockSpec((r, k), lambda i: (i, 0)),
        ],
        out_specs=pl.BlockSpec((r, k), lambda i: (i, 0)),
        out_shape=jax.ShapeDtypeStruct((m, k), jnp.float32),
        compiler_params=pltpu.CompilerParams(
            dimension_semantics=("parallel",),
        ),
    )(inv_tau, alpha, gumbel)
    return out


# speedup vs baseline: 1.0176x; 1.0176x over previous
import functools

import jax
import jax.numpy as jnp
from jax.experimental import pallas as pl
from jax.experimental.pallas import tpu as pltpu


def _onehot_argmax_kernel(inv_tau_ref, a_ref, g_ref, o_ref, *, k):
    x = (a_ref[...] + g_ref[...]) * inv_tau_ref[0]
    m = jnp.max(x, axis=1, keepdims=True)
    ids = jax.lax.broadcasted_iota(jnp.int32, x.shape, 1)
    idx = jnp.min(jnp.where(x >= m, ids, jnp.int32(k)), axis=1, keepdims=True)
    o_ref[...] = (ids == idx).astype(o_ref.dtype)


def kernel(alpha, gumbel, tau):
    m, k = alpha.shape
    r = 16
    inv_tau = (jnp.float32(1.0) / jnp.asarray(tau, jnp.float32)).reshape(1)
    grid = (m // r,)
    out = pl.pallas_call(
        functools.partial(_onehot_argmax_kernel, k=k),
        grid=grid,
        in_specs=[
            pl.BlockSpec(memory_space=pltpu.SMEM),
            pl.BlockSpec((r, k), lambda i: (i, 0)),
            pl.BlockSpec((r, k), lambda i: (i, 0)),
        ],
        out_specs=pl.BlockSpec((r, k), lambda i: (i, 0)),
        out_shape=jax.ShapeDtypeStruct((m, k), jnp.float32),
        compiler_params=pltpu.CompilerParams(
            dimension_semantics=("parallel",),
        ),
    )(inv_tau, alpha, gumbel)
    return out


# speedup vs baseline: 1.0194x; 1.0018x over previous
import functools

import jax
import jax.numpy as jnp
from jax.experimental import pallas as pl
from jax.experimental.pallas import tpu as pltpu

_NBUF = 4
_ROWS = 8


def _ring_kernel(a_hbm, g_hbm, o_hbm, a_buf, g_buf, o_buf, a_sem, g_sem,
                 o_sem, *, m, k):
    nchunks = m // _ROWS

    def a_copy(c, s):
        return pltpu.make_async_copy(
            a_hbm.at[pl.ds(c * _ROWS, _ROWS), :], a_buf.at[s], a_sem.at[s])

    def g_copy(c, s):
        return pltpu.make_async_copy(
            g_hbm.at[pl.ds(c * _ROWS, _ROWS), :], g_buf.at[s], g_sem.at[s])

    def o_copy(c, s):
        return pltpu.make_async_copy(
            o_buf.at[s], o_hbm.at[pl.ds(c * _ROWS, _ROWS), :], o_sem.at[s])

    for s in range(min(_NBUF, nchunks)):
        a_copy(s, s).start()
        g_copy(s, s).start()

    def body(i, carry):
        s = jax.lax.rem(i, _NBUF)
        a_copy(i, s).wait()
        g_copy(i, s).wait()

        @pl.when(i >= _NBUF)
        def _():
            o_copy(i - _NBUF, s).wait()

        x = a_buf[s] + g_buf[s]
        mx = jnp.max(x, axis=1, keepdims=True)
        ids = jax.lax.broadcasted_iota(jnp.int32, x.shape, 1)
        idx = jnp.min(jnp.where(x >= mx, ids, jnp.int32(k)), axis=1,
                      keepdims=True)
        o_buf[s] = (ids == idx).astype(jnp.float32)

        o_copy(i, s).start()

        @pl.when(i + _NBUF < nchunks)
        def _():
            a_copy(i + _NBUF, s).start()
            g_copy(i + _NBUF, s).start()

        return carry

    jax.lax.fori_loop(0, nchunks, body, 0)
    for c in range(max(nchunks - _NBUF, 0), nchunks):
        o_copy(c, c % _NBUF).wait()


def kernel(alpha, gumbel, tau):
    del tau
    m, k = alpha.shape
    buf = lambda: pltpu.VMEM((_NBUF, _ROWS, k), jnp.float32)
    sem = lambda: pltpu.SemaphoreType.DMA((_NBUF,))
    return pl.pallas_call(
        functools.partial(_ring_kernel, m=m, k=k),
        in_specs=[
            pl.BlockSpec(memory_space=pl.ANY),
            pl.BlockSpec(memory_space=pl.ANY),
        ],
        out_specs=pl.BlockSpec(memory_space=pl.ANY),
        out_shape=jax.ShapeDtypeStruct((m, k), jnp.float32),
        scratch_shapes=[buf(), buf(), buf(), sem(), sem(), sem()],
    )(alpha, gumbel)


# speedup vs baseline: 4.0294x; 3.9529x over previous
import functools

import jax
import jax.numpy as jnp
from jax.experimental import pallas as pl
from jax.experimental.pallas import tpu as pltpu

_NBUF = 4


def _pick_chunk(n):
    for c in (800, 200, 8):
        if n % c == 0:
            return c
    return n


def _ring_kernel(a_hbm, g_hbm, o_hbm, a_buf, g_buf, o_buf, ids, macc, iacc,
                 a_sem, g_sem, o_sem, *, n, m, chunk):
    nchunks = n // chunk

    def a_copy(c, s):
        return pltpu.make_async_copy(
            a_hbm.at[pl.ds(c * chunk, chunk), :], a_buf.at[s], a_sem.at[s])

    def g_copy(c, s):
        return pltpu.make_async_copy(
            g_hbm.at[pl.ds(c * chunk, chunk), :], g_buf.at[s], g_sem.at[s])

    def o_copy(c, s):
        return pltpu.make_async_copy(
            o_buf.at[s], o_hbm.at[pl.ds(c * chunk, chunk), :], o_sem.at[s])

    ids[...] = jax.lax.broadcasted_iota(jnp.int32, (chunk, m), 0)
    macc[...] = jnp.full((1, m), -jnp.inf, jnp.float32)
    iacc[...] = jnp.zeros((1, m), jnp.int32)

    for s in range(min(_NBUF, nchunks)):
        a_copy(s, s).start()
        g_copy(s, s).start()

    def body_a(i, carry):
        s = jax.lax.rem(i, _NBUF)
        a_copy(i, s).wait()
        g_copy(i, s).wait()

        x = a_buf[s] + g_buf[s]
        bm = jnp.max(x, axis=0, keepdims=True)
        bi = jnp.min(jnp.where(x >= bm, ids[...], jnp.int32(n)), axis=0,
                     keepdims=True) + i * chunk
        better = bm > macc[...]
        iacc[...] = jnp.where(better, bi, iacc[...])
        macc[...] = jnp.maximum(bm, macc[...])

        @pl.when(i + _NBUF < nchunks)
        def _():
            a_copy(i + _NBUF, s).start()
            g_copy(i + _NBUF, s).start()

        return carry

    jax.lax.fori_loop(0, nchunks, body_a, 0)

    def body_b(i, carry):
        s = jax.lax.rem(i, _NBUF)

        @pl.when(i >= _NBUF)
        def _():
            o_copy(i - _NBUF, s).wait()

        rel = iacc[...] - i * chunk
        o_buf[s] = (ids[...] == rel).astype(jnp.float32)
        o_copy(i, s).start()
        return carry

    jax.lax.fori_loop(0, nchunks, body_b, 0)
    for c in range(max(nchunks - _NBUF, 0), nchunks):
        o_copy(c, c % _NBUF).wait()


def kernel(alpha, gumbel, tau):
    del tau
    mm, kk = alpha.shape
    n, m = kk, mm
    chunk = _pick_chunk(n)
    buf = lambda: pltpu.VMEM((_NBUF, chunk, m), jnp.float32)
    sem = lambda: pltpu.SemaphoreType.DMA((_NBUF,))
    out_t = pl.pallas_call(
        functools.partial(_ring_kernel, n=n, m=m, chunk=chunk),
        in_specs=[
            pl.BlockSpec(memory_space=pl.ANY),
            pl.BlockSpec(memory_space=pl.ANY),
        ],
        out_specs=pl.BlockSpec(memory_space=pl.ANY),
        out_shape=jax.ShapeDtypeStruct((n, m), jnp.float32),
        scratch_shapes=[
            buf(), buf(), buf(),
            pltpu.VMEM((chunk, m), jnp.int32),
            pltpu.VMEM((1, m), jnp.float32),
            pltpu.VMEM((1, m), jnp.int32),
            sem(), sem(), sem(),
        ],
    )(alpha.T, gumbel.T)
    return out_t.T
